# baseline (speedup 1.0000x reference)
import functools

import jax
import jax.numpy as jnp
from jax import lax
from jax.experimental import pallas as pl
from jax.experimental.pallas import tpu as pltpu
from jax.experimental.pallas import tpu_sc as plsc

_B = 16384
_C = 1000
_F = 16
_BM = 4096

_NW = 32
_RW = _B // _NW
_L = 16


def _tc_body(c_ref, xt_ref, eid_ref, cid_ref, cmax_ref):
    c = c_ref[...]
    xt = xt_ref[...]

    xx = jnp.sum(xt * xt, axis=0, keepdims=True)
    yy = jnp.sum(c * c, axis=1, keepdims=True)

    g2 = lax.dot_general(c + c, xt, (((1,), (0,)), ((), ())),
                         preferred_element_type=jnp.float32)
    d = (xx + yy) - g2
    eid_ref[...] = jnp.argmin(d, axis=0)[None, :].astype(jnp.float32)

    fn = xt / jnp.clip(jnp.sqrt(xx), 1e-8, None)
    cn = c / jnp.clip(jnp.sqrt(yy), 1e-8, None)
    cos = lax.dot_general(cn, fn, (((1,), (0,)), ((), ())),
                          preferred_element_type=jnp.float32)
    cmax_ref[...] = jnp.max(cos, axis=0, keepdims=True)
    cid_ref[...] = jnp.argmax(cos, axis=0)[None, :].astype(jnp.float32)


def _tc_partials(feature, centroids):
    xt = feature.T
    eid, cid, cmax = pl.pallas_call(
        _tc_body,
        grid=(_B // _BM,),
        in_specs=[
            pl.BlockSpec((_C, _F), lambda i: (0, 0)),
            pl.BlockSpec((_F, _BM), lambda i: (0, i)),
        ],
        out_specs=[
            pl.BlockSpec((1, _BM), lambda i: (0, i)),
            pl.BlockSpec((1, _BM), lambda i: (0, i)),
            pl.BlockSpec((1, _BM), lambda i: (0, i)),
        ],
        out_shape=[
            jax.ShapeDtypeStruct((1, _B), jnp.float32),
            jax.ShapeDtypeStruct((1, _B), jnp.float32),
            jax.ShapeDtypeStruct((1, _B), jnp.float32),
        ],
    )(centroids, xt)
    return eid[0], cid[0], cmax[0]


def _sc_select_body(eid_hbm, cid_hbm, cmax_hbm, out_hbm,
                    eid_v, cid_v, cmax_v, lab_v):
    wid = lax.axis_index("s") * 2 + lax.axis_index("c")
    base = wid * _RW
    pltpu.sync_copy(eid_hbm.at[pl.ds(base, _RW)], eid_v)
    pltpu.sync_copy(cid_hbm.at[pl.ds(base, _RW)], cid_v)
    pltpu.sync_copy(cmax_hbm.at[pl.ds(base, _RW)], cmax_v)
    for k in range(_RW // _L):
        sl = pl.ds(k * _L, _L)
        e = eid_v[sl]
        ci = cid_v[sl]
        m = cmax_v[sl]
        acc = (ci == e) & (m > 0.85)
        lab_v[sl] = jnp.where(acc, ci, -1.0)
    pltpu.sync_copy(lab_v, out_hbm.at[pl.ds(base, _RW)])


_sc_select = functools.partial(
    pl.kernel,
    out_type=jax.ShapeDtypeStruct((_B,), jnp.float32),
    mesh=plsc.VectorSubcoreMesh(core_axis_name="c", subcore_axis_name="s"),
    scratch_types=[
        pltpu.VMEM((_RW,), jnp.float32),
        pltpu.VMEM((_RW,), jnp.float32),
        pltpu.VMEM((_RW,), jnp.float32),
        pltpu.VMEM((_RW,), jnp.float32),
    ],
)(_sc_select_body)


def kernel(feature, pred, unlabeled_index, centroids):
    del pred, unlabeled_index
    eid, cid, cmax = _tc_partials(feature, centroids)
    return _sc_select(eid, cid, cmax)


# speedup vs baseline: 1.6893x; 1.6893x over previous
import jax
import jax.numpy as jnp
from jax import lax
from jax.experimental import pallas as pl

_B = 16384
_C = 1000
_F = 16
_BM = 4096


def _cluster_body(c_ref, xt_ref, out_ref):
    c = c_ref[...]
    xt = xt_ref[...]

    xx = jnp.sum(xt * xt, axis=0, keepdims=True)
    yy = jnp.sum(c * c, axis=1, keepdims=True)

    g2 = lax.dot_general(c + c, xt, (((1,), (0,)), ((), ())),
                         preferred_element_type=jnp.float32)
    d = (xx + yy) - g2
    e_id = jnp.argmin(d, axis=0)[None, :].astype(jnp.float32)

    fn = xt / jnp.clip(jnp.sqrt(xx), 1e-8, None)
    cn = c / jnp.clip(jnp.sqrt(yy), 1e-8, None)
    cos = lax.dot_general(cn, fn, (((1,), (0,)), ((), ())),
                          preferred_element_type=jnp.float32)
    cmax = jnp.max(cos, axis=0, keepdims=True)
    c_id = jnp.argmax(cos, axis=0)[None, :].astype(jnp.float32)

    accept = (c_id == e_id) & (cmax > 0.85)
    out_ref[...] = jnp.where(accept, c_id, -1.0)


def kernel(feature, pred, unlabeled_index, centroids):
    del pred, unlabeled_index
    xt = feature.T
    out = pl.pallas_call(
        _cluster_body,
        grid=(_B // _BM,),
        in_specs=[
            pl.BlockSpec((_C, _F), lambda i: (0, 0)),
            pl.BlockSpec((_F, _BM), lambda i: (0, i)),
        ],
        out_specs=pl.BlockSpec((1, _BM), lambda i: (0, i)),
        out_shape=jax.ShapeDtypeStruct((1, _B), jnp.float32),
    )(centroids, xt)
    return out[0]


# speedup vs baseline: 1.7223x; 1.0195x over previous
import jax
import jax.numpy as jnp
from jax import lax
from jax.experimental import pallas as pl

_B = 16384
_C = 1000
_F = 16
_BM = 8192


def _cluster_body(c_ref, xt_ref, out_ref):
    c = c_ref[...]
    xt = xt_ref[...]

    xx = jnp.sum(xt * xt, axis=0, keepdims=True)
    yy = jnp.sum(c * c, axis=1, keepdims=True)

    g2 = lax.dot_general(c + c, xt, (((1,), (0,)), ((), ())),
                         preferred_element_type=jnp.float32)
    d = (xx + yy) - g2
    e_id = jnp.argmin(d, axis=0)[None, :].astype(jnp.float32)

    fn = xt / jnp.clip(jnp.sqrt(xx), 1e-8, None)
    cn = c / jnp.clip(jnp.sqrt(yy), 1e-8, None)
    cos = lax.dot_general(cn, fn, (((1,), (0,)), ((), ())),
                          preferred_element_type=jnp.float32)
    cmax = jnp.max(cos, axis=0, keepdims=True)
    c_id = jnp.argmax(cos, axis=0)[None, :].astype(jnp.float32)

    accept = (c_id == e_id) & (cmax > 0.85)
    out_ref[...] = jnp.where(accept, c_id, -1.0)


def kernel(feature, pred, unlabeled_index, centroids):
    del pred, unlabeled_index
    xt = feature.T
    out = pl.pallas_call(
        _cluster_body,
        grid=(_B // _BM,),
        in_specs=[
            pl.BlockSpec((_C, _F), lambda i: (0, 0)),
            pl.BlockSpec((_F, _BM), lambda i: (0, i)),
        ],
        out_specs=pl.BlockSpec((1, _BM), lambda i: (0, i)),
        out_shape=jax.ShapeDtypeStruct((1, _B), jnp.float32),
    )(centroids, xt)
    return out[0]


# speedup vs baseline: 1.8033x; 1.0471x over previous
import jax
import jax.numpy as jnp
from jax import lax
from jax.experimental import pallas as pl

_B = 16384
_C = 1000
_F = 16
_BM = 8192


def _cluster_body(c_ref, xt_ref, out_ref):
    c = c_ref[...]
    xt = xt_ref[...]

    xx = jnp.sum(xt * xt, axis=0, keepdims=True)
    yy = jnp.sum(c * c, axis=1, keepdims=True)

    g2 = lax.dot_general(c + c, xt, (((1,), (0,)), ((), ())),
                         preferred_element_type=jnp.float32)
    d = (xx + yy) - g2
    e_id = jnp.argmin(d, axis=0)[None, :].astype(jnp.float32)

    fn = xt / jnp.clip(jnp.sqrt(xx), 1e-8, None)
    cn = c / jnp.clip(jnp.sqrt(yy), 1e-8, None)
    cos = lax.dot_general(cn, fn, (((1,), (0,)), ((), ())),
                          preferred_element_type=jnp.float32)
    run_v = cos[0:8, :]
    run_s = jnp.zeros((8, _BM), jnp.float32)
    for s in range(1, _C // 8):
        v = cos[8 * s:8 * (s + 1), :]
        m = v > run_v
        run_v = jnp.where(m, v, run_v)
        run_s = jnp.where(m, jnp.float32(s), run_s)
    pos = lax.broadcasted_iota(jnp.int32, (8, 1), 0).astype(jnp.float32)
    run_g = run_s * 8.0 + pos
    vv, gg = run_v, run_g
    for h in (4, 2, 1):
        a_v, b_v = vv[:h, :], vv[h:, :]
        a_g, b_g = gg[:h, :], gg[h:, :]
        m = (b_v > a_v) | ((b_v == a_v) & (b_g < a_g))
        vv = jnp.where(m, b_v, a_v)
        gg = jnp.where(m, b_g, a_g)
    cmax = vv
    c_id = gg

    accept = (c_id == e_id) & (cmax > 0.85)
    out_ref[...] = jnp.where(accept, c_id, -1.0)


def kernel(feature, pred, unlabeled_index, centroids):
    del pred, unlabeled_index
    xt = feature.T
    out = pl.pallas_call(
        _cluster_body,
        grid=(_B // _BM,),
        in_specs=[
            pl.BlockSpec((_C, _F), lambda i: (0, 0)),
            pl.BlockSpec((_F, _BM), lambda i: (0, i)),
        ],
        out_specs=pl.BlockSpec((1, _BM), lambda i: (0, i)),
        out_shape=jax.ShapeDtypeStruct((1, _B), jnp.float32),
    )(centroids, xt)
    return out[0]


# speedup vs baseline: 1.8237x; 1.0113x over previous
import jax
import jax.numpy as jnp
from jax import lax
from jax.experimental import pallas as pl

_B = 16384
_C = 1000
_F = 16
_BM = 8192


def _cluster_body(c_ref, xt_ref, out_ref):
    c = c_ref[...]
    xt = xt_ref[...]

    xx = jnp.sum(xt * xt, axis=0, keepdims=True)
    yy = jnp.sum(c * c, axis=1, keepdims=True)

    g2 = lax.dot_general(c + c, xt, (((1,), (0,)), ((), ())),
                         preferred_element_type=jnp.float32)
    d = (xx + yy) - g2
    ev = d[0:8, :]
    es = jnp.zeros((8, _BM), jnp.float32)
    for s in range(1, _C // 8):
        w = d[8 * s:8 * (s + 1), :]
        me = w < ev
        ev = jnp.where(me, w, ev)
        es = jnp.where(me, jnp.float32(s), es)
    posn = lax.broadcasted_iota(jnp.int32, (8, 1), 0).astype(jnp.float32)
    eg = es * 8.0 + posn
    for h in (4, 2, 1):
        a_v, b_v = ev[:h, :], ev[h:, :]
        a_g, b_g = eg[:h, :], eg[h:, :]
        me = (b_v < a_v) | ((b_v == a_v) & (b_g < a_g))
        ev = jnp.where(me, b_v, a_v)
        eg = jnp.where(me, b_g, a_g)
    e_id = eg

    fn = xt / jnp.clip(jnp.sqrt(xx), 1e-8, None)
    cn = c / jnp.clip(jnp.sqrt(yy), 1e-8, None)
    cos = lax.dot_general(cn, fn, (((1,), (0,)), ((), ())),
                          preferred_element_type=jnp.float32)
    run_v = cos[0:8, :]
    run_s = jnp.zeros((8, _BM), jnp.float32)
    for s in range(1, _C // 8):
        v = cos[8 * s:8 * (s + 1), :]
        m = v > run_v
        run_v = jnp.where(m, v, run_v)
        run_s = jnp.where(m, jnp.float32(s), run_s)
    pos = lax.broadcasted_iota(jnp.int32, (8, 1), 0).astype(jnp.float32)
    run_g = run_s * 8.0 + pos
    vv, gg = run_v, run_g
    for h in (4, 2, 1):
        a_v, b_v = vv[:h, :], vv[h:, :]
        a_g, b_g = gg[:h, :], gg[h:, :]
        m = (b_v > a_v) | ((b_v == a_v) & (b_g < a_g))
        vv = jnp.where(m, b_v, a_v)
        gg = jnp.where(m, b_g, a_g)
    cmax = vv
    c_id = gg

    accept = (c_id == e_id) & (cmax > 0.85)
    out_ref[...] = jnp.where(accept, c_id, -1.0)


def kernel(feature, pred, unlabeled_index, centroids):
    del pred, unlabeled_index
    xt = feature.T
    out = pl.pallas_call(
        _cluster_body,
        grid=(_B // _BM,),
        in_specs=[
            pl.BlockSpec((_C, _F), lambda i: (0, 0)),
            pl.BlockSpec((_F, _BM), lambda i: (0, i)),
        ],
        out_specs=pl.BlockSpec((1, _BM), lambda i: (0, i)),
        out_shape=jax.ShapeDtypeStruct((1, _B), jnp.float32),
    )(centroids, xt)
    return out[0]
